# baseline (speedup 1.0000x reference)
import functools

import jax
import jax.numpy as jnp
from jax import lax
from jax.experimental import pallas as pl

NE = 8


def _wsum_body(w_ref, b_ref, wsum_ref, bsum_ref):
    e = pl.program_id(0)
    colsum = jnp.sum(w_ref[...], axis=0, keepdims=True)
    f = colsum.shape[1]
    i_in = f // NE
    row = lax.broadcasted_iota(jnp.int32, (f, i_in), 0)
    col = lax.broadcasted_iota(jnp.int32, (f, i_in), 1)
    fold = (row % i_in == col).astype(jnp.float32)
    wsum_ref[0] = lax.dot_general(
        colsum, fold, (((1,), (0,)), ((), ())),
        precision=lax.Precision.HIGHEST, preferred_element_type=jnp.float32)

    @pl.when(e == 0)
    def _():
        bsum_ref[...] = jnp.sum(b_ref[...], keepdims=True).reshape(1, 1)


def _pass_a_body(x_ref, wsum_ref, bsum_ref, d_ref, s_ref):
    ti = pl.program_id(1)
    xb = x_ref[0]
    tblk, f = xb.shape
    i_in = f // NE
    ec = d_ref.shape[-1]
    cap = ec // NE
    xw = xb * wsum_ref[...]
    seg_r = lax.broadcasted_iota(jnp.int32, (f, NE), 0)
    seg_c = lax.broadcasted_iota(jnp.int32, (f, NE), 1)
    seg = (seg_r // i_in == seg_c).astype(jnp.float32)
    p = lax.dot_general(xw, seg, (((1,), (0,)), ((), ())),
                        precision=lax.Precision.HIGHEST,
                        preferred_element_type=jnp.float32)
    exp_r = lax.broadcasted_iota(jnp.int32, (NE, ec), 0)
    exp_c = lax.broadcasted_iota(jnp.int32, (NE, ec), 1)
    expand = (exp_c // cap == exp_r).astype(jnp.float32)
    p2 = lax.dot_general(p, expand, (((1,), (0,)), ((), ())),
                         precision=lax.Precision.HIGHEST,
                         preferred_element_type=jnp.float32)
    contrib = jnp.sum(p2 * d_ref[0], axis=0, keepdims=True)

    @pl.when(ti == 0)
    def _():
        s_ref[0] = jnp.broadcast_to(bsum_ref[...], contrib.shape)

    s_ref[0] += contrib


def _pass_b_body(c_ref, s_ref, o_ref):
    y = c_ref[0] * s_ref[0]
    o_ref[0] = jnp.sum(y, axis=1, keepdims=True)


def kernel(x, combine_array, dispatch_mask, weight, bias):
    b, t, f = x.shape
    e, c = dispatch_mask.shape[2], dispatch_mask.shape[3]
    assert e == NE
    ec = e * c
    i_in = f // e
    tblk = 512
    nt = t // tblk

    wsum, bsum = pl.pallas_call(
        _wsum_body,
        grid=(e,),
        in_specs=[
            pl.BlockSpec((i_in, f), lambda g: (g, 0)),
            pl.BlockSpec((1, f), lambda g: (0, 0)),
        ],
        out_specs=[
            pl.BlockSpec((1, 1, i_in), lambda g: (g, 0, 0)),
            pl.BlockSpec((1, 1), lambda g: (0, 0)),
        ],
        out_shape=[
            jax.ShapeDtypeStruct((e, 1, i_in), jnp.float32),
            jax.ShapeDtypeStruct((1, 1), jnp.float32),
        ],
    )(weight, bias.reshape(1, f))

    wsum_flat = wsum.reshape(1, f)
    d2 = dispatch_mask.reshape(b, t, ec)
    c2 = combine_array.reshape(b, t, ec)

    s2 = pl.pallas_call(
        _pass_a_body,
        grid=(b, nt),
        in_specs=[
            pl.BlockSpec((1, tblk, f), lambda gb, gt: (gb, gt, 0)),
            pl.BlockSpec((1, f), lambda gb, gt: (0, 0)),
            pl.BlockSpec((1, 1), lambda gb, gt: (0, 0)),
            pl.BlockSpec((1, tblk, ec), lambda gb, gt: (gb, gt, 0)),
        ],
        out_specs=pl.BlockSpec((1, 1, ec), lambda gb, gt: (gb, 0, 0)),
        out_shape=jax.ShapeDtypeStruct((b, 1, ec), jnp.float32),
    )(x, wsum_flat, bsum, d2)

    out = pl.pallas_call(
        _pass_b_body,
        grid=(b, nt),
        in_specs=[
            pl.BlockSpec((1, tblk, ec), lambda gb, gt: (gb, gt, 0)),
            pl.BlockSpec((1, 1, ec), lambda gb, gt: (gb, 0, 0)),
        ],
        out_specs=pl.BlockSpec((1, tblk, 1), lambda gb, gt: (gb, gt, 0)),
        out_shape=jax.ShapeDtypeStruct((b, t, 1), jnp.float32),
    )(c2, s2)

    return out.reshape(b, t)


# speedup vs baseline: 3.3241x; 3.3241x over previous
import functools

import jax
import jax.numpy as jnp
from jax import lax
from jax.experimental import pallas as pl

NE = 8


def _wsum_body(w_ref, b_ref, wsum_ref, bsum_ref):
    e = pl.program_id(0)
    colsum = jnp.sum(w_ref[...], axis=0, keepdims=True)
    f = colsum.shape[1]
    i_in = f // NE
    row = lax.broadcasted_iota(jnp.int32, (f, i_in), 0)
    col = lax.broadcasted_iota(jnp.int32, (f, i_in), 1)
    fold = (row % i_in == col).astype(jnp.float32)
    wsum_ref[0] = lax.dot_general(
        colsum, fold, (((1,), (0,)), ((), ())),
        precision=lax.Precision.HIGHEST, preferred_element_type=jnp.float32)

    @pl.when(e == 0)
    def _():
        bsum_ref[...] = jnp.sum(b_ref[...], keepdims=True).reshape(1, 1)


def _pass_a_body(x_ref, wsum_ref, bsum_ref, d_ref, s_ref, *, nt):
    g = pl.program_id(0)
    xb = x_ref[...]
    f = xb.shape[1]
    i_in = f // NE
    cap = d_ref.shape[2]
    xw = xb * wsum_ref[...]
    seg_r = lax.broadcasted_iota(jnp.int32, (f, NE), 0)
    seg_c = lax.broadcasted_iota(jnp.int32, (f, NE), 1)
    seg = (seg_r // i_in == seg_c).astype(jnp.float32)
    p = lax.dot_general(xw, seg, (((1,), (0,)), ((), ())),
                        precision=lax.Precision.HIGHEST,
                        preferred_element_type=jnp.float32)
    incs = [
        jnp.sum(d_ref[:, e, :] * p[:, e:e + 1], axis=0, keepdims=True)
        for e in range(NE)
    ]
    inc = jnp.concatenate(incs, axis=0)

    @pl.when(g % nt == 0)
    def _():
        s_ref[0] = jnp.broadcast_to(bsum_ref[...], (NE, cap))

    s_ref[0] += inc


def _pass_b_body(c_ref, s_ref, o_ref):
    sl = s_ref[0]
    acc = c_ref[:, 0, :] * sl[0:1, :]
    for e in range(1, NE):
        acc += c_ref[:, e, :] * sl[e:e + 1, :]
    o_ref[...] = jnp.sum(acc, axis=1, keepdims=True)


def kernel(x, combine_array, dispatch_mask, weight, bias):
    b, t, f = x.shape
    e, c = dispatch_mask.shape[2], dispatch_mask.shape[3]
    assert e == NE
    i_in = f // e
    tblk = 512
    bt = b * t
    nt = t // tblk
    nbt = bt // tblk

    wsum, bsum = pl.pallas_call(
        _wsum_body,
        grid=(e,),
        in_specs=[
            pl.BlockSpec((i_in, f), lambda g: (g, 0)),
            pl.BlockSpec((1, f), lambda g: (0, 0)),
        ],
        out_specs=[
            pl.BlockSpec((1, 1, i_in), lambda g: (g, 0, 0)),
            pl.BlockSpec((1, 1), lambda g: (0, 0)),
        ],
        out_shape=[
            jax.ShapeDtypeStruct((e, 1, i_in), jnp.float32),
            jax.ShapeDtypeStruct((1, 1), jnp.float32),
        ],
    )(weight, bias.reshape(1, f))

    wsum_flat = wsum.reshape(1, f)
    x2 = x.reshape(bt, f)
    d3 = dispatch_mask.reshape(bt, e, c)
    c3 = combine_array.reshape(bt, e, c)

    s = pl.pallas_call(
        functools.partial(_pass_a_body, nt=nt),
        grid=(nbt,),
        in_specs=[
            pl.BlockSpec((tblk, f), lambda g: (g, 0)),
            pl.BlockSpec((1, f), lambda g: (0, 0)),
            pl.BlockSpec((1, 1), lambda g: (0, 0)),
            pl.BlockSpec((tblk, e, c), lambda g: (g, 0, 0)),
        ],
        out_specs=pl.BlockSpec((1, e, c), lambda g, nt=nt: (g // nt, 0, 0)),
        out_shape=jax.ShapeDtypeStruct((b, e, c), jnp.float32),
    )(x2, wsum_flat, bsum, d3)

    out = pl.pallas_call(
        _pass_b_body,
        grid=(nbt,),
        in_specs=[
            pl.BlockSpec((tblk, e, c), lambda g: (g, 0, 0)),
            pl.BlockSpec((1, e, c), lambda g, nt=nt: (g // nt, 0, 0)),
        ],
        out_specs=pl.BlockSpec((tblk, 1), lambda g: (g, 0)),
        out_shape=jax.ShapeDtypeStruct((bt, 1), jnp.float32),
    )(c3, s)

    return out.reshape(b, t)
